# Initial kernel scaffold; baseline (speedup 1.0000x reference)
import jax
import jax.numpy as jnp
from jax.experimental import pallas as pl


def kernel(x, edge_index, edge_attr, u, batch, W1a, b1a, g1, be1, W1b, b1b, W2a, b2a, g2a, be2a, W2b, b2b, g2b, be2b, W2c, b2c):
    raise NotImplementedError("write your pallas kernel here")


# speedup vs baseline: 3.2305x; 3.2305x over previous
import functools

import jax
import jax.numpy as jnp
from jax import lax
from jax.experimental import pallas as pl
from jax.experimental.pallas import tpu as pltpu
from jax.experimental.pallas import tpu_sc as plsc

NC = 2
NS = 16
NW = NC * NS
L = 16

CE = 128
CN = 80


def _mm_bias(A, W, b, block_rows):
    M, K = A.shape
    Hout = W.shape[1]
    assert M % block_rows == 0

    def body(a_ref, w_ref, b_ref, o_ref):
        o_ref[...] = (
            jnp.dot(a_ref[...], w_ref[...], preferred_element_type=jnp.float32, precision=lax.Precision.HIGHEST)
            + b_ref[...]
        )

    return pl.pallas_call(
        body,
        grid=(M // block_rows,),
        in_specs=[
            pl.BlockSpec((block_rows, K), lambda i: (i, 0)),
            pl.BlockSpec((K, Hout), lambda i: (0, 0)),
            pl.BlockSpec((1, Hout), lambda i: (0, 0)),
        ],
        out_specs=pl.BlockSpec((block_rows, Hout), lambda i: (i, 0)),
        out_shape=jax.ShapeDtypeStruct((M, Hout), jnp.float32),
    )(A, W, b.reshape(1, Hout))


def _sc_aggregate(xa, eb, edge_index, batch, xc, G):
    N, HID = xa.shape
    E = eb.shape[0]
    assert E % CE == 0 and N % CN == 0
    nchunk_e = E // CE
    nchunk_n = N // CN
    rows_per_tile = G // NS

    mesh = plsc.VectorSubcoreMesh(core_axis_name="c", subcore_axis_name="s")

    def body(xa_hbm, eb_hbm, ei_hbm, batch_hbm, xc_hbm,
             t_out, gx_out, q_out, cnt_out,
             row_v, col_v, ge_v, nb_v,
             eb_buf, xa_buf, cnt_v, qstage, stage, xbuf,
             sharedT, sharedGX, gsem, gsem2):
        cid = lax.axis_index("c")
        sid = lax.axis_index("s")
        wid = sid * NC + cid

        zf = jnp.zeros((L,), jnp.float32)
        ones = jnp.ones((L,), jnp.float32)

        for j in range(G // L):
            cnt_v[pl.ds(j * L, L)] = zf

        def zrow(i, _):
            for k in range(HID // L):
                stage[i, pl.ds(k * L, L)] = zf
            return 0

        lax.fori_loop(0, rows_per_tile, zrow, 0)
        pltpu.sync_copy(stage, sharedT.at[pl.ds(sid * rows_per_tile, rows_per_tile)])
        pltpu.sync_copy(stage, sharedGX.at[pl.ds(sid * rows_per_tile, rows_per_tile)])
        plsc.subcore_barrier()

        n_iter_e = (nchunk_e - wid + NW - 1) // NW

        def echunk(i, q):
            c = wid + i * NW
            base = c * CE
            pltpu.sync_copy(ei_hbm.at[0, pl.ds(base, CE)], row_v)
            pltpu.sync_copy(ei_hbm.at[1, pl.ds(base, CE)], col_v)
            gath = pltpu.async_copy(xa_hbm.at[row_v], xa_buf, gsem)
            gath2 = pltpu.async_copy(batch_hbm.at[col_v], ge_v, gsem2)
            pltpu.sync_copy(eb_hbm.at[pl.ds(base, CE)], eb_buf)
            gath.wait()

            def ebody(e, qq):
                nq = []
                for k in range(HID // L):
                    z = eb_buf[e, pl.ds(k * L, L)] + xa_buf[e, pl.ds(k * L, L)]
                    hh = jnp.maximum(z, 0.01 * z)
                    eb_buf[e, pl.ds(k * L, L)] = hh
                    nq.append(qq[k] + hh * hh)
                return tuple(nq)

            q = lax.fori_loop(0, CE, ebody, q)
            gath2.wait()
            for j in range(CE // L):
                gv = ge_v[pl.ds(j * L, L)]
                plsc.addupdate_scatter(cnt_v, [gv], ones)
            pltpu.sync_copy(eb_buf, sharedT.at[ge_v], add=True)
            return q

        q0 = tuple(jnp.zeros((L,), jnp.float32) for _ in range(HID // L))
        q = lax.fori_loop(0, n_iter_e, echunk, q0)
        for k in range(HID // L):
            qstage[pl.ds(k * L, L)] = q[k]

        n_iter_n = (nchunk_n - wid + NW - 1) // NW

        def nchunk(i, _):
            c = wid + i * NW
            base = c * CN
            gb = pltpu.async_copy(batch_hbm.at[pl.ds(base, CN)], nb_v, gsem2)
            pltpu.sync_copy(xc_hbm.at[pl.ds(base, CN)], xbuf)
            gb.wait()
            pltpu.sync_copy(xbuf, sharedGX.at[nb_v], add=True)
            return 0

        lax.fori_loop(0, n_iter_n, nchunk, 0)

        pltpu.sync_copy(qstage, q_out.at[wid])
        pltpu.sync_copy(cnt_v, cnt_out.at[wid])
        plsc.subcore_barrier()
        r0 = sid * rows_per_tile
        pltpu.sync_copy(sharedT.at[pl.ds(r0, rows_per_tile)], stage)
        pltpu.sync_copy(stage, t_out.at[cid, pl.ds(r0, rows_per_tile)])
        pltpu.sync_copy(sharedGX.at[pl.ds(r0, rows_per_tile)], stage)
        pltpu.sync_copy(stage, gx_out.at[cid, pl.ds(r0, rows_per_tile)])

    fn = pl.kernel(
        body,
        out_type=(
            jax.ShapeDtypeStruct((NC, G, HID), jnp.float32),
            jax.ShapeDtypeStruct((NC, G, HID), jnp.float32),
            jax.ShapeDtypeStruct((NW, HID), jnp.float32),
            jax.ShapeDtypeStruct((NW, G), jnp.float32),
        ),
        mesh=mesh,
        scratch_types=[
            pltpu.VMEM((CE,), jnp.int32),
            pltpu.VMEM((CE,), jnp.int32),
            pltpu.VMEM((CE,), jnp.int32),
            pltpu.VMEM((CN,), jnp.int32),
            pltpu.VMEM((CE, HID), jnp.float32),
            pltpu.VMEM((CE, HID), jnp.float32),
            pltpu.VMEM((G,), jnp.float32),
            pltpu.VMEM((HID,), jnp.float32),
            pltpu.VMEM((G // NS, HID), jnp.float32),
            pltpu.VMEM((CN, HID), jnp.float32),
            pltpu.VMEM_SHARED((G, HID), jnp.float32),
            pltpu.VMEM_SHARED((G, HID), jnp.float32),
            pltpu.SemaphoreType.DMA,
            pltpu.SemaphoreType.DMA,
        ],
        compiler_params=pltpu.CompilerParams(needs_layout_passes=False),
    )
    return fn(xa, eb, edge_index, batch, xc)


def _graph_mlp(Tp, GXp, Qp, CNTp, E, W1b, b1b, g1, be1,
               W2a, b2a, g2a, be2a, W2b, b2b, g2b, be2b, W2c, b2c):
    G = Tp.shape[1]
    HID = W1b.shape[0]
    HN = W2a.shape[0] - HID
    NGo = W2c.shape[1]

    def body(tp_ref, gxp_ref, qp_ref, cntp_ref,
             w1b_ref, b1b_ref, g1_ref, be1_ref,
             w2a_ref, b2a_ref, g2a_ref, be2a_ref,
             w2b_ref, b2b_ref, g2b_ref, be2b_ref,
             w2c_ref, b2c_ref, o_ref):
        T = tp_ref[0] + tp_ref[1]
        cnt_col = lax.dot_general(
            cntp_ref[...], jnp.ones((NW, 1), jnp.float32),
            (((0,), (0,)), ((), ())), preferred_element_type=jnp.float32, precision=lax.Precision.HIGHEST)
        gxh = gxp_ref[0] + gxp_ref[1]
        Q = jnp.sum(qp_ref[...], axis=0, keepdims=True)
        S = jnp.sum(T, axis=0, keepdims=True)
        m = S / E
        v = Q / E - m * m
        a = g1_ref[...] * lax.rsqrt(v + 1e-5)
        cconst = be1_ref[...] - m * a
        bias_row = (
            jnp.dot(cconst, w1b_ref[...], preferred_element_type=jnp.float32, precision=lax.Precision.HIGHEST)
            + b1b_ref[...]
        )
        go = (
            jnp.dot(T * a, w1b_ref[...], preferred_element_type=jnp.float32, precision=lax.Precision.HIGHEST)
            + cnt_col * bias_row
        )

        def leaky(t):
            return jnp.maximum(t, 0.01 * t)

        def bn(t, g, b):
            mm = jnp.mean(t, axis=0, keepdims=True)
            vv = jnp.mean(t * t, axis=0, keepdims=True) - mm * mm
            return g * (t - mm) * lax.rsqrt(vv + 1e-5) + b

        h = (
            gxh
            + jnp.dot(go, w2a_ref[HN:], preferred_element_type=jnp.float32, precision=lax.Precision.HIGHEST)
            + b2a_ref[...]
        )
        h = bn(leaky(h), g2a_ref[...], be2a_ref[...])
        h = jnp.dot(h, w2b_ref[...], preferred_element_type=jnp.float32, precision=lax.Precision.HIGHEST) + b2b_ref[...]
        h = bn(leaky(h), g2b_ref[...], be2b_ref[...])
        h = jnp.dot(h, w2c_ref[...], preferred_element_type=jnp.float32, precision=lax.Precision.HIGHEST) + b2c_ref[...]
        o_ref[...] = h

    r = lambda t: t.reshape(1, -1)
    return pl.pallas_call(
        body,
        out_shape=jax.ShapeDtypeStruct((G, NGo), jnp.float32),
    )(Tp, GXp, Qp, CNTp,
      W1b, r(b1b), r(g1), r(be1),
      W2a, r(b2a), r(g2a), r(be2a),
      W2b, r(b2b), r(g2b), r(be2b),
      W2c, r(b2c))


def kernel(x, edge_index, edge_attr, u, batch,
           W1a, b1a, g1, be1, W1b, b1b,
           W2a, b2a, g2a, be2a, W2b, b2b, g2b, be2b, W2c, b2c):
    N, HN = x.shape
    E = edge_index.shape[1]
    G = u.shape[0]
    HID = W1a.shape[1]

    xa = _mm_bias(x, W1a[:HN], jnp.zeros((HID,), jnp.float32), 2000)
    eb = _mm_bias(edge_attr, W1a[HN:], b1a, 8000)
    xc = _mm_bias(x, W2a[:HN], jnp.zeros((HID,), jnp.float32), 2000)
    Tp, GXp, Qp, CNTp = _sc_aggregate(xa, eb, edge_index, batch, xc, G)
    return _graph_mlp(Tp, GXp, Qp, CNTp, E, W1b, b1b, g1, be1,
                      W2a, b2a, g2a, be2a, W2b, b2b, g2b, be2b, W2c, b2c)


# speedup vs baseline: 4.6680x; 1.4450x over previous
import numpy as np

import jax
import jax.numpy as jnp
from jax import lax
from jax.experimental import pallas as pl
from jax.experimental.pallas import tpu as pltpu
from jax.experimental.pallas import tpu_sc as plsc

NC = 2
NS = 16
NW = NC * NS
L = 16

CE = 128
CN = 80


def _dot3(a, w):
    a_hi = a.astype(jnp.bfloat16)
    a_lo = (a - a_hi.astype(jnp.float32)).astype(jnp.bfloat16)
    w_hi = w.astype(jnp.bfloat16)
    w_lo = (w - w_hi.astype(jnp.float32)).astype(jnp.bfloat16)
    d = lambda x, y: jnp.dot(x, y, preferred_element_type=jnp.float32)
    return d(a_hi, w_hi) + d(a_hi, w_lo) + d(a_lo, w_hi)


def _mm_bias(A, W, b, block_rows):
    M, K = A.shape
    Hout = W.shape[1]
    assert M % block_rows == 0

    def body(a_ref, w_ref, b_ref, o_ref):
        o_ref[...] = _dot3(a_ref[...], w_ref[...]) + b_ref[...]

    return pl.pallas_call(
        body,
        grid=(M // block_rows,),
        in_specs=[
            pl.BlockSpec((block_rows, K), lambda i: (i, 0)),
            pl.BlockSpec((K, Hout), lambda i: (0, 0)),
            pl.BlockSpec((1, Hout), lambda i: (0, 0)),
        ],
        out_specs=pl.BlockSpec((block_rows, Hout), lambda i: (i, 0)),
        out_shape=jax.ShapeDtypeStruct((M, Hout), jnp.float32),
    )(A, W, b.reshape(1, Hout))


def _mm_two(A, W1, W2, block_rows):
    M, K = A.shape
    Hout = W1.shape[1]
    assert M % block_rows == 0

    def body(a_ref, w1_ref, w2_ref, o1_ref, o2_ref):
        a = a_ref[...]
        o1_ref[...] = _dot3(a, w1_ref[...])
        o2_ref[...] = _dot3(a, w2_ref[...])

    return pl.pallas_call(
        body,
        grid=(M // block_rows,),
        in_specs=[
            pl.BlockSpec((block_rows, K), lambda i: (i, 0)),
            pl.BlockSpec((K, Hout), lambda i: (0, 0)),
            pl.BlockSpec((K, Hout), lambda i: (0, 0)),
        ],
        out_specs=[
            pl.BlockSpec((block_rows, Hout), lambda i: (i, 0)),
            pl.BlockSpec((block_rows, Hout), lambda i: (i, 0)),
        ],
        out_shape=[
            jax.ShapeDtypeStruct((M, Hout), jnp.float32),
            jax.ShapeDtypeStruct((M, Hout), jnp.float32),
        ],
    )(A, W1, W2)


def _sc_aggregate(xa, eb, edge_index, batch, xc, G):
    N, HID = xa.shape
    E = eb.shape[0]
    assert E % CE == 0 and N % CN == 0
    nchunk_e = E // CE
    nchunk_n = N // CN
    rows_per_tile = G // NS

    mesh = plsc.VectorSubcoreMesh(core_axis_name="c", subcore_axis_name="s")

    npair_total = nchunk_e // 2
    assert npair_total * 2 == nchunk_e

    def body(xa_hbm, eb_hbm, ei_hbm, batch_hbm, xc_hbm,
             t_out, gx_out, q_out, cnt_out,
             row0, col0, row1, col1, ge0, ge1, nb_v,
             eb0, eb1, xa0, xa1, h0, h1,
             cnt_v, qstage, stage, xbuf,
             sharedT, sharedGX,
             isem0, isem1, esem0, esem1, gsem0, gsem1,
             bsem0, bsem1, ssem0, ssem1, nsem):
        cid = lax.axis_index("c")
        sid = lax.axis_index("s")
        wid = sid * NC + cid

        zf = jnp.zeros((L,), jnp.float32)
        ones = jnp.ones((L,), jnp.float32)

        for j in range(G // L):
            cnt_v[pl.ds(j * L, L)] = zf

        def zrow(i, _):
            for k in range(HID // L):
                stage[i, pl.ds(k * L, L)] = zf
            return 0

        lax.fori_loop(0, rows_per_tile, zrow, 0)
        pltpu.sync_copy(stage, sharedT.at[pl.ds(sid * rows_per_tile, rows_per_tile)])
        pltpu.sync_copy(stage, sharedGX.at[pl.ds(sid * rows_per_tile, rows_per_tile)])
        plsc.subcore_barrier()

        qn, rn = divmod(npair_total, NW)
        npairs = qn + jnp.where(wid < rn, 1, 0)
        start = jnp.where(
            wid < rn, wid * 2 * (qn + 1), rn * 2 * (qn + 1) + (wid - rn) * 2 * qn
        )

        def compute(ebb, xab, hb, q):
            def ebody(i, qq):
                nq = list(qq)
                for u in range(2):
                    e = 2 * i + u
                    for k in range(HID // L):
                        z = ebb[e, pl.ds(k * L, L)] + xab[e, pl.ds(k * L, L)]
                        hh = jnp.maximum(z, 0.01 * z)
                        hb[e, pl.ds(k * L, L)] = hh
                        nq[k] = nq[k] + hh * hh
                return tuple(nq)

            return lax.fori_loop(0, CE // 2, ebody, q)

        def cnt_add(ge):
            for j in range(CE // L):
                gv = ge[pl.ds(j * L, L)]
                plsc.addupdate_scatter(cnt_v, [gv], ones)

        base_p = start * CE
        pltpu.sync_copy(ei_hbm.at[0, pl.ds(base_p, CE)], row0)
        pltpu.sync_copy(ei_hbm.at[1, pl.ds(base_p, CE)], col0)
        pltpu.async_copy(ei_hbm.at[0, pl.ds(base_p + CE, CE)], row1, isem1)
        pltpu.async_copy(ei_hbm.at[1, pl.ds(base_p + CE, CE)], col1, isem1)
        pltpu.async_copy(eb_hbm.at[pl.ds(base_p, CE)], eb0, esem0)
        pltpu.async_copy(xa_hbm.at[row0], xa0, gsem0)

        def pair_body(j, q):
            base0 = (start + 2 * j) * CE
            base1 = base0 + CE
            base2 = base0 + 2 * CE
            base3 = base0 + 3 * CE
            more = j + 1 < npairs

            pltpu.make_async_copy(ei_hbm.at[0, pl.ds(base1, CE)], row1, isem1).wait()
            pltpu.make_async_copy(ei_hbm.at[1, pl.ds(base1, CE)], col1, isem1).wait()
            pltpu.async_copy(eb_hbm.at[pl.ds(base1, CE)], eb1, esem1)
            pltpu.async_copy(xa_hbm.at[row1], xa1, gsem1)

            @pl.when(j > 0)
            def _():
                pltpu.make_async_copy(h0, sharedT.at[ge0], ssem0).wait()

            pltpu.async_copy(batch_hbm.at[col0], ge0, bsem0)
            pltpu.make_async_copy(eb_hbm.at[pl.ds(base0, CE)], eb0, esem0).wait()
            pltpu.make_async_copy(xa_hbm.at[row0], xa0, gsem0).wait()
            pltpu.make_async_copy(batch_hbm.at[col0], ge0, bsem0).wait()

            @pl.when(more)
            def _():
                pltpu.async_copy(ei_hbm.at[0, pl.ds(base2, CE)], row0, isem0)
                pltpu.async_copy(ei_hbm.at[1, pl.ds(base2, CE)], col0, isem0)

            q = compute(eb0, xa0, h0, q)
            cnt_add(ge0)
            pltpu.async_copy(h0, sharedT.at[ge0], ssem0, add=True)

            @pl.when(more)
            def _():
                pltpu.make_async_copy(ei_hbm.at[0, pl.ds(base2, CE)], row0, isem0).wait()
                pltpu.make_async_copy(ei_hbm.at[1, pl.ds(base2, CE)], col0, isem0).wait()
                pltpu.async_copy(eb_hbm.at[pl.ds(base2, CE)], eb0, esem0)
                pltpu.async_copy(xa_hbm.at[row0], xa0, gsem0)

            @pl.when(j > 0)
            def _():
                pltpu.make_async_copy(h1, sharedT.at[ge1], ssem1).wait()

            pltpu.async_copy(batch_hbm.at[col1], ge1, bsem1)
            pltpu.make_async_copy(eb_hbm.at[pl.ds(base1, CE)], eb1, esem1).wait()
            pltpu.make_async_copy(xa_hbm.at[row1], xa1, gsem1).wait()
            pltpu.make_async_copy(batch_hbm.at[col1], ge1, bsem1).wait()

            @pl.when(more)
            def _():
                pltpu.async_copy(ei_hbm.at[0, pl.ds(base3, CE)], row1, isem1)
                pltpu.async_copy(ei_hbm.at[1, pl.ds(base3, CE)], col1, isem1)

            q = compute(eb1, xa1, h1, q)
            cnt_add(ge1)
            pltpu.async_copy(h1, sharedT.at[ge1], ssem1, add=True)
            return q

        q0 = tuple(jnp.zeros((L,), jnp.float32) for _ in range(HID // L))
        q = lax.fori_loop(0, npairs, pair_body, q0)
        pltpu.make_async_copy(h0, sharedT.at[ge0], ssem0).wait()
        pltpu.make_async_copy(h1, sharedT.at[ge1], ssem1).wait()
        for k in range(HID // L):
            qstage[pl.ds(k * L, L)] = q[k]

        n_iter_n = (nchunk_n - wid + NW - 1) // NW

        def nchunk(i, _):
            c = wid + i * NW
            base = c * CN
            gb = pltpu.async_copy(batch_hbm.at[pl.ds(base, CN)], nb_v, nsem)
            pltpu.sync_copy(xc_hbm.at[pl.ds(base, CN)], xbuf)
            gb.wait()
            pltpu.sync_copy(xbuf, sharedGX.at[nb_v], add=True)
            return 0

        lax.fori_loop(0, n_iter_n, nchunk, 0)

        pltpu.sync_copy(qstage, q_out.at[wid])
        pltpu.sync_copy(cnt_v, cnt_out.at[wid])
        plsc.subcore_barrier()
        r0 = sid * rows_per_tile
        pltpu.sync_copy(sharedT.at[pl.ds(r0, rows_per_tile)], stage)
        pltpu.sync_copy(stage, t_out.at[cid, pl.ds(r0, rows_per_tile)])
        pltpu.sync_copy(sharedGX.at[pl.ds(r0, rows_per_tile)], stage)
        pltpu.sync_copy(stage, gx_out.at[cid, pl.ds(r0, rows_per_tile)])

    fn = pl.kernel(
        body,
        out_type=(
            jax.ShapeDtypeStruct((NC, G, HID), jnp.float32),
            jax.ShapeDtypeStruct((NC, G, HID), jnp.float32),
            jax.ShapeDtypeStruct((NW, HID), jnp.float32),
            jax.ShapeDtypeStruct((NW, G), jnp.float32),
        ),
        mesh=mesh,
        scratch_types=[
            pltpu.VMEM((CE,), jnp.int32),
            pltpu.VMEM((CE,), jnp.int32),
            pltpu.VMEM((CE,), jnp.int32),
            pltpu.VMEM((CE,), jnp.int32),
            pltpu.VMEM((CE,), jnp.int32),
            pltpu.VMEM((CE,), jnp.int32),
            pltpu.VMEM((CN,), jnp.int32),
            pltpu.VMEM((CE, HID), jnp.float32),
            pltpu.VMEM((CE, HID), jnp.float32),
            pltpu.VMEM((CE, HID), jnp.float32),
            pltpu.VMEM((CE, HID), jnp.float32),
            pltpu.VMEM((CE, HID), jnp.float32),
            pltpu.VMEM((CE, HID), jnp.float32),
            pltpu.VMEM((G,), jnp.float32),
            pltpu.VMEM((HID,), jnp.float32),
            pltpu.VMEM((G // NS, HID), jnp.float32),
            pltpu.VMEM((CN, HID), jnp.float32),
            pltpu.VMEM_SHARED((G, HID), jnp.float32),
            pltpu.VMEM_SHARED((G, HID), jnp.float32),
        ] + [pltpu.SemaphoreType.DMA] * 11,
        compiler_params=pltpu.CompilerParams(needs_layout_passes=False),
    )
    return fn(xa, eb, edge_index, batch, xc)


def _graph_mlp(Tp, GXp, Qp, CNTp, E, W1b, b1b, g1, be1,
               W2a, b2a, g2a, be2a, W2b, b2b, g2b, be2b, W2c, b2c):
    G = Tp.shape[1]
    HID = W1b.shape[0]
    HN = W2a.shape[0] - HID
    NGo = W2c.shape[1]

    def body(tp_ref, gxp_ref, qp_ref, cntp_ref,
             w1b_ref, b1b_ref, g1_ref, be1_ref,
             w2a_ref, b2a_ref, g2a_ref, be2a_ref,
             w2b_ref, b2b_ref, g2b_ref, be2b_ref,
             w2c_ref, b2c_ref, o_ref):
        T = tp_ref[0] + tp_ref[1]
        cnt_col = lax.dot_general(
            cntp_ref[...], jnp.ones((NW, 1), jnp.float32),
            (((0,), (0,)), ((), ())), preferred_element_type=jnp.float32, precision=lax.Precision.HIGHEST)
        gxh = gxp_ref[0] + gxp_ref[1]
        Q = jnp.sum(qp_ref[...], axis=0, keepdims=True)
        S = jnp.sum(T, axis=0, keepdims=True)
        m = S / E
        v = Q / E - m * m
        a = g1_ref[...] * lax.rsqrt(v + 1e-5)
        cconst = be1_ref[...] - m * a
        bias_row = (
            jnp.dot(cconst, w1b_ref[...], preferred_element_type=jnp.float32, precision=lax.Precision.HIGHEST)
            + b1b_ref[...]
        )
        go = (
            jnp.dot(T * a, w1b_ref[...], preferred_element_type=jnp.float32, precision=lax.Precision.HIGHEST)
            + cnt_col * bias_row
        )

        def leaky(t):
            return jnp.maximum(t, 0.01 * t)

        def bn(t, g, b):
            mm = jnp.mean(t, axis=0, keepdims=True)
            vv = jnp.mean(t * t, axis=0, keepdims=True) - mm * mm
            return g * (t - mm) * lax.rsqrt(vv + 1e-5) + b

        h = (
            gxh
            + jnp.dot(go, w2a_ref[HN:], preferred_element_type=jnp.float32, precision=lax.Precision.HIGHEST)
            + b2a_ref[...]
        )
        h = bn(leaky(h), g2a_ref[...], be2a_ref[...])
        h = jnp.dot(h, w2b_ref[...], preferred_element_type=jnp.float32, precision=lax.Precision.HIGHEST) + b2b_ref[...]
        h = bn(leaky(h), g2b_ref[...], be2b_ref[...])
        h = jnp.dot(h, w2c_ref[...], preferred_element_type=jnp.float32, precision=lax.Precision.HIGHEST) + b2c_ref[...]
        o_ref[...] = h

    r = lambda t: t.reshape(1, -1)
    return pl.pallas_call(
        body,
        out_shape=jax.ShapeDtypeStruct((G, NGo), jnp.float32),
    )(Tp, GXp, Qp, CNTp,
      W1b, r(b1b), r(g1), r(be1),
      W2a, r(b2a), r(g2a), r(be2a),
      W2b, r(b2b), r(g2b), r(be2b),
      W2c, r(b2c))


def kernel(x, edge_index, edge_attr, u, batch,
           W1a, b1a, g1, be1, W1b, b1b,
           W2a, b2a, g2a, be2a, W2b, b2b, g2b, be2b, W2c, b2c):
    N, HN = x.shape
    E = edge_index.shape[1]
    G = u.shape[0]
    HID = W1a.shape[1]

    xa, xc = _mm_two(x, W1a[:HN], W2a[:HN], 2000)
    eb = _mm_bias(edge_attr, W1a[HN:], b1a, 8000)
    Tp, GXp, Qp, CNTp = _sc_aggregate(xa, eb, edge_index, batch, xc, G)
    return _graph_mlp(Tp, GXp, Qp, CNTp, E, W1b, b1b, g1, be1,
                      W2a, b2a, g2a, be2a, W2b, b2b, g2b, be2b, W2c, b2c)
